# Initial kernel scaffold; baseline (speedup 1.0000x reference)
import jax
import jax.numpy as jnp
from jax.experimental import pallas as pl


def kernel(x, edge_index, batch, W1, b1, W2, b2):
    raise NotImplementedError("write your pallas kernel here")


# speedup vs baseline: 10.7604x; 10.7604x over previous
import functools

import jax
import jax.numpy as jnp
from jax import lax
from jax.experimental import pallas as pl
from jax.experimental.pallas import tpu as pltpu
from jax.experimental.pallas import tpu_sc as plsc

N = 10000
E = 160000
F_IN = 514
H = 128
C = 2
G = 64

NC = 2
NS = 16
NW = NC * NS
L = 16

NPAD = 10240
RPW = NPAD // NS
KP = 640
WIN = 128
EPW = 5120
NWIN = EPW // WIN
ET_PAD = NW * EPW
W2P = 16

_mesh = lambda: plsc.VectorSubcoreMesh(
    core_axis_name="c", subcore_axis_name="s", num_cores=NC, num_subcores=NS)


def _deg_kernel(dstp, erows, zrows):
    @functools.partial(
        pl.kernel,
        out_type=jax.ShapeDtypeStruct((NC, NPAD, L), jnp.float32),
        mesh=_mesh(),
        scratch_types=[
            pltpu.VMEM((WIN,), jnp.int32),
            pltpu.VMEM((WIN, L), jnp.float32),
            pltpu.VMEM_SHARED((NPAD, L), jnp.float32),
        ],
    )
    def k(dst_hbm, erows_hbm, zrows_hbm, out_hbm, didx, ones_v, acc):
        cid = lax.axis_index("c")
        sid = lax.axis_index("s")
        wid = cid * NS + sid
        pltpu.sync_copy(erows_hbm, ones_v)
        pltpu.sync_copy(zrows_hbm, acc.at[pl.ds(sid * RPW, RPW)])
        plsc.subcore_barrier()

        @pl.loop(0, NWIN)
        def _w(j):
            base = wid * EPW + j * WIN
            pltpu.sync_copy(dst_hbm.at[pl.ds(base, WIN)], didx)
            pltpu.sync_copy(ones_v, acc.at[didx], add=True)

        plsc.subcore_barrier()
        pltpu.sync_copy(acc.at[pl.ds(sid * RPW, RPW)],
                        out_hbm.at[cid, pl.ds(sid * RPW, RPW)])

    return k(dstp, erows, zrows)


@functools.lru_cache(maxsize=None)
def _make_agg(width):
    params = None
    if width % 128 != 0:
        params = pltpu.CompilerParams(use_tc_tiling_on_sc=False)

    @functools.partial(
        pl.kernel,
        out_type=jax.ShapeDtypeStruct((NC, NPAD, width), jnp.float32),
        mesh=_mesh(),
        compiler_params=params,
        scratch_types=[
            pltpu.VMEM((WIN,), jnp.int32),
            pltpu.VMEM((WIN,), jnp.int32),
            pltpu.VMEM((WIN, width), jnp.float32),
            pltpu.VMEM_SHARED((NPAD, width), jnp.float32),
            pltpu.SemaphoreType.DMA,
        ],
    )
    def k(table_hbm, src_hbm, dst_hbm, zrows_hbm, out_hbm,
          sidx, didx, rows, acc, sem):
        cid = lax.axis_index("c")
        sid = lax.axis_index("s")
        wid = cid * NS + sid
        pltpu.sync_copy(zrows_hbm, acc.at[pl.ds(sid * RPW, RPW)])
        plsc.subcore_barrier()

        @pl.loop(0, NWIN)
        def _w(j):
            base = wid * EPW + j * WIN
            pltpu.sync_copy(src_hbm.at[pl.ds(base, WIN)], sidx)
            pltpu.async_copy(table_hbm.at[sidx], rows, sem).wait()
            pltpu.sync_copy(dst_hbm.at[pl.ds(base, WIN)], didx)
            pltpu.sync_copy(rows, acc.at[didx], add=True)

        plsc.subcore_barrier()
        pltpu.sync_copy(acc.at[pl.ds(sid * RPW, RPW)],
                        out_hbm.at[cid, pl.ds(sid * RPW, RPW)])

    return k


def _mm_body(x_ref, w_ref, o_ref):
    o_ref[...] = jnp.dot(x_ref[...], w_ref[...],
                         preferred_element_type=jnp.float32)


def _matmul(xp, w1p):
    bm = 256
    return pl.pallas_call(
        _mm_body,
        grid=(NPAD // bm,),
        in_specs=[
            pl.BlockSpec((bm, KP), lambda i: (i, 0)),
            pl.BlockSpec((KP, H), lambda i: (0, 0)),
        ],
        out_specs=pl.BlockSpec((bm, H), lambda i: (i, 0)),
        out_shape=jax.ShapeDtypeStruct((NPAD, H), jnp.float32),
    )(xp, w1p)


def _scale_body(degp_ref, h1_ref, h1s_ref, dinv_ref):
    deg = degp_ref[0, :, 0:1] + degp_ref[1, :, 0:1] + 1.0
    dinv = lax.rsqrt(deg)
    dinv_ref[...] = dinv
    h1s_ref[...] = h1_ref[...] * dinv


def _scale(degp, h1):
    return pl.pallas_call(
        _scale_body,
        out_shape=(
            jax.ShapeDtypeStruct((NPAD, H), jnp.float32),
            jax.ShapeDtypeStruct((NPAD, 1), jnp.float32),
        ),
    )(degp, h1)


def _layer2_body(part_ref, h1s_ref, dinv_ref, b1_ref, w2_ref, zs_ref):
    dinv = dinv_ref[...]
    pre = (part_ref[0] + part_ref[1] + h1s_ref[...]) * dinv + b1_ref[...]
    rel = jnp.maximum(pre, 0.0)
    z = jnp.dot(rel, w2_ref[...], preferred_element_type=jnp.float32)
    zs_ref[...] = z * dinv


def _layer2(part1, h1s, dinv, b1r, w2p):
    return pl.pallas_call(
        _layer2_body,
        out_shape=jax.ShapeDtypeStruct((NPAD, W2P), jnp.float32),
    )(part1, h1s, dinv, b1r, w2p)


def _pool_body(part_ref, zs_ref, dinv_ref, b2_ref, batch_ref, out_ref):
    vals = (part_ref[0] + part_ref[1] + zs_ref[...]) * dinv_ref[...] + b2_ref[...]
    gids = lax.broadcasted_iota(jnp.int32, (NPAD, G), 1)
    onehot = (batch_ref[...] == gids).astype(jnp.float32)
    sums = lax.dot_general(onehot, vals, (((0,), (0,)), ((), ())),
                           preferred_element_type=jnp.float32)
    out_ref[...] = sums[:, :C] / jnp.maximum(sums[:, C:C + 1], 1.0)


def _pool(part2, zs16, dinv, b2r, batchp):
    return pl.pallas_call(
        _pool_body,
        out_shape=jax.ShapeDtypeStruct((G, C), jnp.float32),
    )(part2, zs16, dinv, b2r, batchp)


def kernel(x, edge_index, batch, W1, b1, W2, b2):
    f32 = jnp.float32
    src = edge_index[0]
    dst = edge_index[1]
    npad_e = ET_PAD - E
    pad_idx = N + (jnp.arange(npad_e, dtype=jnp.int32) % 64)
    srcp = jnp.concatenate([src, pad_idx])
    dstp = jnp.concatenate([dst, pad_idx])

    xp = jnp.pad(x, ((0, NPAD - N), (0, KP - F_IN)))
    w1p = jnp.pad(W1, ((0, KP - F_IN), (0, 0)))
    w2p = jnp.pad(W2, ((0, 0), (0, W2P - C)))
    b1r = b1.reshape(1, H)
    b2r = jnp.concatenate(
        [b2, jnp.ones((1,), f32), jnp.zeros((W2P - C - 1,), f32)]).reshape(1, W2P)
    batchp = jnp.pad(batch, (0, NPAD - N), constant_values=G).reshape(NPAD, 1)

    erows = jnp.tile(
        jnp.concatenate([jnp.ones((1,), f32), jnp.zeros((L - 1,), f32)]),
        (WIN, 1)).reshape(WIN, L)
    zrows128 = jnp.zeros((RPW, H), f32)
    zrows16 = jnp.zeros((RPW, W2P), f32)

    h1 = _matmul(xp, w1p)
    degp = _deg_kernel(dstp, erows, zrows16)
    h1s, dinv = _scale(degp, h1)
    part1 = _make_agg(H)(h1s, srcp, dstp, zrows128)
    zs16 = _layer2(part1, h1s, dinv, b1r, w2p)
    part2 = _make_agg(W2P)(zs16, srcp, dstp, zrows16)
    return _pool(part2, zs16, dinv, b2r, batchp)


# speedup vs baseline: 14.8555x; 1.3806x over previous
import functools

import jax
import jax.numpy as jnp
from jax import lax
from jax.experimental import pallas as pl
from jax.experimental.pallas import tpu as pltpu
from jax.experimental.pallas import tpu_sc as plsc

N = 10000
E = 160000
F_IN = 514
H = 128
C = 2
G = 64

NC = 2
NS = 16
NW = NC * NS
L = 16

NPAD = 10240
RPW = NPAD // NS
KP = 640
WIN = 128
EPW = 5120
NWIN = EPW // WIN
ET_PAD = NW * EPW
W2P = 16

_mesh = lambda: plsc.VectorSubcoreMesh(
    core_axis_name="c", subcore_axis_name="s", num_cores=NC, num_subcores=NS)


def _drain(dummy_src, dst_ref, sem):
    pltpu.make_async_copy(dummy_src, dst_ref, sem).wait()


def _deg_kernel(dst3, erows, zrows):
    @functools.partial(
        pl.kernel,
        out_type=jax.ShapeDtypeStruct((NC, NPAD, L), jnp.float32),
        mesh=_mesh(),
        scratch_types=[
            pltpu.VMEM((NWIN, WIN), jnp.int32),
            pltpu.VMEM((WIN, L), jnp.float32),
            pltpu.VMEM_SHARED((NPAD, L), jnp.float32),
        ],
    )
    def k(dst_hbm, erows_hbm, zrows_hbm, out_hbm, didx, ones_v, acc):
        cid = lax.axis_index("c")
        sid = lax.axis_index("s")
        wid = cid * NS + sid
        pltpu.sync_copy(erows_hbm, ones_v)
        pltpu.sync_copy(dst_hbm.at[wid], didx)
        pltpu.sync_copy(zrows_hbm, acc.at[pl.ds(sid * RPW, RPW)])
        plsc.subcore_barrier()

        @pl.loop(0, NWIN)
        def _w(j):
            pltpu.sync_copy(ones_v, acc.at[didx.at[j]], add=True)

        plsc.subcore_barrier()
        pltpu.sync_copy(acc.at[pl.ds(sid * RPW, RPW)],
                        out_hbm.at[cid, pl.ds(sid * RPW, RPW)])

    return k(dst3, erows, zrows)


@functools.lru_cache(maxsize=None)
def _make_agg(width, tc_tiling):
    nb = 2
    scratch = [
        pltpu.VMEM((NWIN, WIN), jnp.int32),
        pltpu.VMEM((NWIN, WIN), jnp.int32),
    ] + [pltpu.VMEM((WIN, width), jnp.float32) for _ in range(nb)]
    scratch.append(pltpu.VMEM_SHARED((NPAD, width), jnp.float32))
    scratch += [pltpu.SemaphoreType.DMA] * nb

    @functools.partial(
        pl.kernel,
        out_type=jax.ShapeDtypeStruct((NC, NPAD, width), jnp.float32),
        mesh=_mesh(),
        scratch_types=scratch,
        compiler_params=pltpu.CompilerParams(use_tc_tiling_on_sc=tc_tiling),
    )
    def k(table_hbm, src_hbm, dst_hbm, zrows_hbm, out_hbm, sidx, didx, *rest):
        rows = rest[:nb]
        acc = rest[nb]
        gsem = rest[nb + 1:]
        tab = table_hbm
        cid = lax.axis_index("c")
        sid = lax.axis_index("s")
        wid = cid * NS + sid
        pltpu.sync_copy(src_hbm.at[wid], sidx)
        pltpu.sync_copy(dst_hbm.at[wid], didx)
        pltpu.sync_copy(zrows_hbm, acc.at[pl.ds(sid * RPW, RPW)])
        plsc.subcore_barrier()

        for b in range(nb):
            pltpu.async_copy(tab.at[sidx.at[b]], rows[b], gsem[b])

        @pl.loop(0, NWIN // nb)
        def _w(q):
            for b in range(nb):
                j = q * nb + b
                _drain(table_hbm.at[pl.ds(0, WIN)], rows[b], gsem[b])
                pltpu.sync_copy(rows[b], acc.at[didx.at[j]], add=True)

                @pl.when(j + nb < NWIN)
                def _():
                    pltpu.async_copy(tab.at[sidx.at[j + nb]], rows[b], gsem[b])

        plsc.subcore_barrier()
        pltpu.sync_copy(acc.at[pl.ds(sid * RPW, RPW)],
                        out_hbm.at[cid, pl.ds(sid * RPW, RPW)])

    return k


def _mm_body(x_ref, w_ref, o_ref):
    o_ref[...] = jnp.dot(x_ref[...], w_ref[...],
                         preferred_element_type=jnp.float32)


def _matmul(xp, w1p):
    bm = 256
    return pl.pallas_call(
        _mm_body,
        grid=(NPAD // bm,),
        in_specs=[
            pl.BlockSpec((bm, KP), lambda i: (i, 0)),
            pl.BlockSpec((KP, H), lambda i: (0, 0)),
        ],
        out_specs=pl.BlockSpec((bm, H), lambda i: (i, 0)),
        out_shape=jax.ShapeDtypeStruct((NPAD, H), jnp.float32),
    )(xp, w1p)


def _scale_body(degp_ref, h1_ref, h1s_ref, dinv_ref):
    deg = degp_ref[0, :, 0:1] + degp_ref[1, :, 0:1] + 1.0
    dinv = lax.rsqrt(deg)
    dinv_ref[...] = dinv
    h1s_ref[...] = h1_ref[...] * dinv


def _scale(degp, h1):
    return pl.pallas_call(
        _scale_body,
        out_shape=(
            jax.ShapeDtypeStruct((NPAD, H), jnp.float32),
            jax.ShapeDtypeStruct((NPAD, 1), jnp.float32),
        ),
    )(degp, h1)


def _layer2_body(part_ref, h1s_ref, dinv_ref, b1_ref, w2_ref, zs_ref):
    dinv = dinv_ref[...]
    pre = (part_ref[0] + part_ref[1] + h1s_ref[...]) * dinv + b1_ref[...]
    rel = jnp.maximum(pre, 0.0)
    z = jnp.dot(rel, w2_ref[...], preferred_element_type=jnp.float32)
    zs_ref[...] = z * dinv


def _layer2(part1, h1s, dinv, b1r, w2p):
    return pl.pallas_call(
        _layer2_body,
        out_shape=jax.ShapeDtypeStruct((NPAD, W2P), jnp.float32),
    )(part1, h1s, dinv, b1r, w2p)


def _pool_body(part_ref, zs_ref, dinv_ref, b2_ref, batch_ref, out_ref):
    vals = (part_ref[0] + part_ref[1] + zs_ref[...]) * dinv_ref[...] + b2_ref[...]
    gids = lax.broadcasted_iota(jnp.int32, (NPAD, G), 1)
    onehot = (batch_ref[...] == gids).astype(jnp.float32)
    sums = lax.dot_general(onehot, vals, (((0,), (0,)), ((), ())),
                           preferred_element_type=jnp.float32)
    out_ref[...] = sums[:, :C] / jnp.maximum(sums[:, C:C + 1], 1.0)


def _pool(part2, zs16, dinv, b2r, batchp):
    return pl.pallas_call(
        _pool_body,
        out_shape=jax.ShapeDtypeStruct((G, C), jnp.float32),
    )(part2, zs16, dinv, b2r, batchp)


def kernel(x, edge_index, batch, W1, b1, W2, b2):
    f32 = jnp.float32
    src = edge_index[0]
    dst = edge_index[1]
    npad_e = ET_PAD - E
    pad_idx = N + (jnp.arange(npad_e, dtype=jnp.int32) % 64)
    src3 = jnp.concatenate([src, pad_idx]).reshape(NW, NWIN, WIN)
    dst3 = jnp.concatenate([dst, pad_idx]).reshape(NW, NWIN, WIN)

    xp = jnp.pad(x, ((0, NPAD - N), (0, KP - F_IN)))
    w1p = jnp.pad(W1, ((0, KP - F_IN), (0, 0)))
    w2p = jnp.pad(W2, ((0, 0), (0, W2P - C)))
    b1r = b1.reshape(1, H)
    b2r = jnp.concatenate(
        [b2, jnp.ones((1,), f32), jnp.zeros((W2P - C - 1,), f32)]).reshape(1, W2P)
    batchp = jnp.pad(batch, (0, NPAD - N), constant_values=G).reshape(NPAD, 1)

    erows = jnp.tile(
        jnp.concatenate([jnp.ones((1,), f32), jnp.zeros((L - 1,), f32)]),
        (WIN, 1)).reshape(WIN, L)
    zrows128 = jnp.zeros((RPW, H), f32)
    zrows16 = jnp.zeros((RPW, W2P), f32)

    h1 = _matmul(xp, w1p)
    degp = _deg_kernel(dst3, erows, zrows16)
    h1s, dinv = _scale(degp, h1)
    part1 = _make_agg(H, True)(h1s, src3, dst3, zrows128)
    zs16 = _layer2(part1, h1s, dinv, b1r, w2p)
    part2 = _make_agg(W2P, False)(zs16, src3, dst3, zrows16)
    return _pool(part2, zs16, dinv, b2r, batchp)


# speedup vs baseline: 20.6904x; 1.3928x over previous
import functools

import jax
import jax.numpy as jnp
from jax import lax
from jax.experimental import pallas as pl
from jax.experimental.pallas import tpu as pltpu
from jax.experimental.pallas import tpu_sc as plsc

N = 10000
E = 160000
F_IN = 514
H = 128
C = 2
G = 64

NC = 2
NS = 16
NW = NC * NS
L = 16

NPAD = 10240
RPW = NPAD // NS
WIN = 128
EPW = 5120
NWIN = EPW // WIN
ET_PAD = NW * EPW
W2P = 16

_mesh = lambda: plsc.VectorSubcoreMesh(
    core_axis_name="c", subcore_axis_name="s", num_cores=NC, num_subcores=NS)


def _drain(dummy_src, dst_ref, sem):
    pltpu.make_async_copy(dummy_src, dst_ref, sem).wait()


def _deg_kernel(dst3, erows, zrows):
    @functools.partial(
        pl.kernel,
        out_type=jax.ShapeDtypeStruct((NC, NPAD, L), jnp.float32),
        mesh=_mesh(),
        scratch_types=[
            pltpu.VMEM((NWIN, WIN), jnp.int32),
            pltpu.VMEM((WIN, L), jnp.float32),
            pltpu.VMEM_SHARED((NPAD, L), jnp.float32),
        ],
    )
    def k(dst_hbm, erows_hbm, zrows_hbm, out_hbm, didx, ones_v, acc):
        cid = lax.axis_index("c")
        sid = lax.axis_index("s")
        wid = cid * NS + sid
        pltpu.sync_copy(erows_hbm, ones_v)
        pltpu.sync_copy(dst_hbm.at[wid], didx)
        pltpu.sync_copy(zrows_hbm, acc.at[pl.ds(sid * RPW, RPW)])
        plsc.subcore_barrier()

        @pl.loop(0, NWIN)
        def _w(j):
            pltpu.sync_copy(ones_v, acc.at[didx.at[j]], add=True)

        plsc.subcore_barrier()
        pltpu.sync_copy(acc.at[pl.ds(sid * RPW, RPW)],
                        out_hbm.at[cid, pl.ds(sid * RPW, RPW)])

    return k(dst3, erows, zrows)


@functools.lru_cache(maxsize=None)
def _make_agg(width, tc_tiling):
    nb = 2
    scratch = [
        pltpu.VMEM((NWIN, WIN), jnp.int32),
        pltpu.VMEM((NWIN, WIN), jnp.int32),
    ] + [pltpu.VMEM((WIN, width), jnp.float32) for _ in range(nb)]
    scratch.append(pltpu.VMEM_SHARED((NPAD, width), jnp.float32))
    scratch += [pltpu.SemaphoreType.DMA] * nb

    @functools.partial(
        pl.kernel,
        out_type=jax.ShapeDtypeStruct((NC, NPAD, width), jnp.float32),
        mesh=_mesh(),
        scratch_types=scratch,
        compiler_params=pltpu.CompilerParams(use_tc_tiling_on_sc=tc_tiling),
    )
    def k(table_hbm, src_hbm, dst_hbm, zrows_hbm, out_hbm, sidx, didx, *rest):
        rows = rest[:nb]
        acc = rest[nb]
        gsem = rest[nb + 1:]
        tab = table_hbm
        cid = lax.axis_index("c")
        sid = lax.axis_index("s")
        wid = cid * NS + sid
        pltpu.sync_copy(src_hbm.at[wid], sidx)
        pltpu.sync_copy(dst_hbm.at[wid], didx)
        pltpu.sync_copy(zrows_hbm, acc.at[pl.ds(sid * RPW, RPW)])
        plsc.subcore_barrier()

        for b in range(nb):
            pltpu.async_copy(tab.at[sidx.at[b]], rows[b], gsem[b])

        @pl.loop(0, NWIN // nb)
        def _w(q):
            for b in range(nb):
                j = q * nb + b
                _drain(table_hbm.at[pl.ds(0, WIN)], rows[b], gsem[b])
                pltpu.sync_copy(rows[b], acc.at[didx.at[j]], add=True)

                @pl.when(j + nb < NWIN)
                def _():
                    pltpu.async_copy(tab.at[sidx.at[j + nb]], rows[b], gsem[b])

        plsc.subcore_barrier()
        pltpu.sync_copy(acc.at[pl.ds(sid * RPW, RPW)],
                        out_hbm.at[cid, pl.ds(sid * RPW, RPW)])

    return k


def _mm_body(x_ref, w_ref, o_ref):
    o_ref[...] = jnp.dot(x_ref[...], w_ref[...],
                         preferred_element_type=jnp.float32)


def _matmul(x, w1):
    bm = 256
    return pl.pallas_call(
        _mm_body,
        grid=(NPAD // bm,),
        in_specs=[
            pl.BlockSpec((bm, F_IN), lambda i: (i, 0)),
            pl.BlockSpec((F_IN, H), lambda i: (0, 0)),
        ],
        out_specs=pl.BlockSpec((bm, H), lambda i: (i, 0)),
        out_shape=jax.ShapeDtypeStruct((NPAD, H), jnp.float32),
    )(x, w1)


def _scale_body(degp_ref, h1_ref, h1s_ref, dinv_ref):
    deg = degp_ref[0, :, 0:1] + degp_ref[1, :, 0:1] + 1.0
    dinv = lax.rsqrt(deg)
    dinv_ref[...] = dinv
    h1s_ref[...] = h1_ref[...] * dinv


def _scale(degp, h1):
    return pl.pallas_call(
        _scale_body,
        out_shape=(
            jax.ShapeDtypeStruct((NPAD, H), jnp.float32),
            jax.ShapeDtypeStruct((NPAD, 1), jnp.float32),
        ),
    )(degp, h1)


def _layer2_body(part_ref, h1s_ref, dinv_ref, b1_ref, w2_ref, zs_ref):
    dinv = dinv_ref[...]
    pre = (part_ref[0] + part_ref[1] + h1s_ref[...]) * dinv + b1_ref[...]
    rel = jnp.maximum(pre, 0.0)
    z = jnp.dot(rel, w2_ref[...], preferred_element_type=jnp.float32)
    zs_ref[...] = z * dinv


def _layer2(part1, h1s, dinv, b1r, w2p):
    return pl.pallas_call(
        _layer2_body,
        out_shape=jax.ShapeDtypeStruct((NPAD, W2P), jnp.float32),
    )(part1, h1s, dinv, b1r, w2p)


def _pool_body(part_ref, zs_ref, dinv_ref, b2_ref, batch_ref, out_ref):
    vals = (part_ref[0] + part_ref[1] + zs_ref[...]) * dinv_ref[...] + b2_ref[...]
    vals = jnp.where(batch_ref[...] < G, vals, 0.0)
    gids = lax.broadcasted_iota(jnp.int32, (NPAD, G), 1)
    onehot = (batch_ref[...] == gids).astype(jnp.float32)
    sums = lax.dot_general(onehot, vals, (((0,), (0,)), ((), ())),
                           preferred_element_type=jnp.float32)
    out_ref[...] = sums[:, :C] / jnp.maximum(sums[:, C:C + 1], 1.0)


def _pool(part2, zs16, dinv, b2r, batchp):
    return pl.pallas_call(
        _pool_body,
        out_shape=jax.ShapeDtypeStruct((G, C), jnp.float32),
    )(part2, zs16, dinv, b2r, batchp)


def kernel(x, edge_index, batch, W1, b1, W2, b2):
    f32 = jnp.float32
    src = edge_index[0]
    dst = edge_index[1]
    npad_e = ET_PAD - E
    pad_idx = N + (jnp.arange(npad_e, dtype=jnp.int32) % 64)
    src3 = jnp.concatenate([src, pad_idx]).reshape(NW, NWIN, WIN)
    dst3 = jnp.concatenate([dst, pad_idx]).reshape(NW, NWIN, WIN)

    w2p = jnp.pad(W2, ((0, 0), (0, W2P - C)))
    b1r = b1.reshape(1, H)
    b2r = jnp.concatenate(
        [b2, jnp.ones((1,), f32), jnp.zeros((W2P - C - 1,), f32)]).reshape(1, W2P)
    batchp = jnp.pad(batch, (0, NPAD - N), constant_values=G).reshape(NPAD, 1)

    erows = jnp.tile(
        jnp.concatenate([jnp.ones((1,), f32), jnp.zeros((L - 1,), f32)]),
        (WIN, 1)).reshape(WIN, L)
    zrows128 = jnp.zeros((RPW, H), f32)
    zrows16 = jnp.zeros((RPW, W2P), f32)

    h1 = _matmul(x, W1)
    degp = _deg_kernel(dst3, erows, zrows16)
    h1s, dinv = _scale(degp, h1)
    part1 = _make_agg(H, True)(h1s, src3, dst3, zrows128)
    zs16 = _layer2(part1, h1s, dinv, b1r, w2p)
    part2 = _make_agg(W2P, False)(zs16, src3, dst3, zrows16)
    return _pool(part2, zs16, dinv, b2r, batchp)


# speedup vs baseline: 21.6406x; 1.0459x over previous
import functools

import jax
import jax.numpy as jnp
from jax import lax
from jax.experimental import pallas as pl
from jax.experimental.pallas import tpu as pltpu
from jax.experimental.pallas import tpu_sc as plsc

N = 10000
E = 160000
F_IN = 514
H = 128
C = 2
G = 64

NC = 2
NS = 16
NW = NC * NS
L = 16

NPAD = 10240
RPW = NPAD // NS
WIN = 128
EPW = 5120
NWIN = EPW // WIN
ET_PAD = NW * EPW
W2P = 16

_mesh = lambda: plsc.VectorSubcoreMesh(
    core_axis_name="c", subcore_axis_name="s", num_cores=NC, num_subcores=NS)


def _drain(dummy_src, dst_ref, sem):
    pltpu.make_async_copy(dummy_src, dst_ref, sem).wait()


def _deg_kernel(dst3, erows, zrows):
    @functools.partial(
        pl.kernel,
        out_type=jax.ShapeDtypeStruct((NC, NPAD, L), jnp.float32),
        mesh=_mesh(),
        scratch_types=[
            pltpu.VMEM((NWIN, WIN), jnp.int32),
            pltpu.VMEM((WIN, L), jnp.float32),
            pltpu.VMEM_SHARED((NPAD, L), jnp.float32),
        ],
    )
    def k(dst_hbm, erows_hbm, zrows_hbm, out_hbm, didx, ones_v, acc):
        cid = lax.axis_index("c")
        sid = lax.axis_index("s")
        wid = cid * NS + sid
        pltpu.sync_copy(erows_hbm, ones_v)
        pltpu.sync_copy(dst_hbm.at[wid], didx)
        pltpu.sync_copy(zrows_hbm, acc.at[pl.ds(sid * RPW, RPW)])
        plsc.subcore_barrier()

        @pl.loop(0, NWIN)
        def _w(j):
            pltpu.sync_copy(ones_v, acc.at[didx.at[j]], add=True)

        plsc.subcore_barrier()
        pltpu.sync_copy(acc.at[pl.ds(sid * RPW, RPW)],
                        out_hbm.at[cid, pl.ds(sid * RPW, RPW)])

    return k(dst3, erows, zrows)


@functools.lru_cache(maxsize=None)
def _make_agg(width, tc_tiling):
    nb = 2
    scratch = [
        pltpu.VMEM((NWIN, WIN), jnp.int32),
        pltpu.VMEM((NWIN, WIN), jnp.int32),
    ] + [pltpu.VMEM((WIN, width), jnp.float32) for _ in range(nb)]
    scratch.append(pltpu.VMEM_SHARED((NPAD, width), jnp.float32))
    scratch += [pltpu.SemaphoreType.DMA] * nb

    @functools.partial(
        pl.kernel,
        out_type=jax.ShapeDtypeStruct((NC, NPAD, width), jnp.float32),
        mesh=_mesh(),
        scratch_types=scratch,
        compiler_params=pltpu.CompilerParams(use_tc_tiling_on_sc=tc_tiling),
    )
    def k(table_hbm, src_hbm, dst_hbm, zrows_hbm, out_hbm, sidx, didx, *rest):
        rows = rest[:nb]
        acc = rest[nb]
        gsem = rest[nb + 1:]
        tab = table_hbm
        cid = lax.axis_index("c")
        sid = lax.axis_index("s")
        wid = cid * NS + sid
        pltpu.sync_copy(src_hbm.at[wid], sidx)
        pltpu.sync_copy(dst_hbm.at[wid], didx)
        pltpu.sync_copy(zrows_hbm, acc.at[pl.ds(sid * RPW, RPW)])
        plsc.subcore_barrier()

        for b in range(nb):
            pltpu.async_copy(tab.at[sidx.at[b]], rows[b], gsem[b])

        @pl.loop(0, NWIN // nb)
        def _w(q):
            for b in range(nb):
                j = q * nb + b
                _drain(table_hbm.at[pl.ds(0, WIN)], rows[b], gsem[b])
                pltpu.sync_copy(rows[b], acc.at[didx.at[j]], add=True)

                @pl.when(j + nb < NWIN)
                def _():
                    pltpu.async_copy(tab.at[sidx.at[j + nb]], rows[b], gsem[b])

        plsc.subcore_barrier()
        pltpu.sync_copy(acc.at[pl.ds(sid * RPW, RPW)],
                        out_hbm.at[cid, pl.ds(sid * RPW, RPW)])

    return k


def _mm_body(x_ref, w_ref, o_ref):
    o_ref[...] = jnp.dot(x_ref[...], w_ref[...],
                         preferred_element_type=jnp.float32)


def _matmul(x, w1):
    bm = 400
    return pl.pallas_call(
        _mm_body,
        grid=(N // bm,),
        in_specs=[
            pl.BlockSpec((bm, F_IN), lambda i: (i, 0)),
            pl.BlockSpec((F_IN, H), lambda i: (0, 0)),
        ],
        out_specs=pl.BlockSpec((bm, H), lambda i: (i, 0)),
        out_shape=jax.ShapeDtypeStruct((N, H), jnp.float32),
    )(x, w1)


def _scale_body(degp_ref, h1_ref, h1s_ref, dinv_ref):
    deg = (degp_ref[0, :N, 0:1] + degp_ref[1, :N, 0:1]) + 1.0
    dinv = lax.rsqrt(deg)
    dinv_ref[...] = dinv
    h1s_ref[...] = h1_ref[...] * dinv


def _scale(degp, h1):
    return pl.pallas_call(
        _scale_body,
        out_shape=(
            jax.ShapeDtypeStruct((N, H), jnp.float32),
            jax.ShapeDtypeStruct((N, 1), jnp.float32),
        ),
    )(degp, h1)


def _layer2_body(part_ref, h1s_ref, dinv_ref, b1_ref, w2_ref, zs_ref):
    dinv = dinv_ref[...]
    pre = (part_ref[0, :N] + part_ref[1, :N] + h1s_ref[...]) * dinv + b1_ref[...]
    rel = jnp.maximum(pre, 0.0)
    z = jnp.dot(rel, w2_ref[...], preferred_element_type=jnp.float32)
    zs_ref[...] = z * dinv


def _layer2(part1, h1s, dinv, b1r, w2p):
    return pl.pallas_call(
        _layer2_body,
        out_shape=jax.ShapeDtypeStruct((N, W2P), jnp.float32),
    )(part1, h1s, dinv, b1r, w2p)


def _pool_body(part_ref, zs_ref, dinv_ref, b2_ref, batch_ref, out_ref):
    vals = ((part_ref[0, :N] + part_ref[1, :N] + zs_ref[...])
            * dinv_ref[...] + b2_ref[...])
    gids = lax.broadcasted_iota(jnp.int32, (N, G), 1)
    onehot = (batch_ref[...] == gids).astype(jnp.float32)
    sums = lax.dot_general(onehot, vals, (((0,), (0,)), ((), ())),
                           preferred_element_type=jnp.float32)
    out_ref[...] = sums[:, :C] / jnp.maximum(sums[:, C:C + 1], 1.0)


def _pool(part2, zs16, dinv, b2r, batchp):
    return pl.pallas_call(
        _pool_body,
        out_shape=jax.ShapeDtypeStruct((G, C), jnp.float32),
    )(part2, zs16, dinv, b2r, batchp)


def kernel(x, edge_index, batch, W1, b1, W2, b2):
    f32 = jnp.float32
    src = edge_index[0]
    dst = edge_index[1]
    npad_e = ET_PAD - E
    pad_lo = jnp.arange(npad_e, dtype=jnp.int32) % 64
    src3 = jnp.concatenate([src, pad_lo]).reshape(NW, NWIN, WIN)
    dst3 = jnp.concatenate([dst, N + pad_lo]).reshape(NW, NWIN, WIN)

    w2p = jnp.pad(W2, ((0, 0), (0, W2P - C)))
    b1r = b1.reshape(1, H)
    b2r = jnp.concatenate(
        [b2, jnp.ones((1,), f32), jnp.zeros((W2P - C - 1,), f32)]).reshape(1, W2P)
    batchp = batch.reshape(N, 1)

    erows = jnp.tile(
        jnp.concatenate([jnp.ones((1,), f32), jnp.zeros((L - 1,), f32)]),
        (WIN, 1)).reshape(WIN, L)
    zrows128 = jnp.zeros((RPW, H), f32)
    zrows16 = jnp.zeros((RPW, W2P), f32)

    h1 = _matmul(x, W1)
    degp = _deg_kernel(dst3, erows, zrows16)
    h1s, dinv = _scale(degp, h1)
    part1 = _make_agg(H, True)(h1s, src3, dst3, zrows128)
    zs16 = _layer2(part1, h1s, dinv, b1r, w2p)
    part2 = _make_agg(W2P, False)(zs16, src3, dst3, zrows16)
    return _pool(part2, zs16, dinv, b2r, batchp)
